# baseline (speedup 1.0000x reference)
import functools

import jax
import jax.numpy as jnp
from jax.experimental import pallas as pl
from jax.experimental.pallas import tpu as pltpu

H = 128
H2 = 64
N = 10000
E = 320000
KT = 80000
L = 200
NK = 100000


def _gelu(x):
    return jax.nn.gelu(x, approximate=False)


def _gelu_k(x):
    return 0.5 * x * (1.0 + jax.lax.erf(x * 0.7071067811865476))


def _layernorm(x, g, b):
    mu = x.mean(-1, keepdims=True)
    v = ((x - mu) ** 2).mean(-1, keepdims=True)
    return (x - mu) / jnp.sqrt(v + 1e-5) * g + b


def _kw_attn_body(kemb_ref, k1w_ref, k1b_ref, k2w_ref, k2b_ref, aq_ref, w_ref):
    rows = kemb_ref[...]
    h = _gelu_k(rows @ k1w_ref[...] + k1b_ref[...])
    ak = h @ k2w_ref[...] + k2b_ref[...]
    w_ref[...] = jnp.sum(ak * aq_ref[...], axis=1)


def _kw_attention(kemb, k1w, k1b, k2w, k2b, aq):
    B = 10240
    grid = pl.cdiv(NK, B)
    return pl.pallas_call(
        _kw_attn_body,
        grid=(grid,),
        in_specs=[
            pl.BlockSpec((B, H2), lambda i: (i, 0)),
            pl.BlockSpec((H2, H2), lambda i: (0, 0)),
            pl.BlockSpec((H2,), lambda i: (0,)),
            pl.BlockSpec((H2, H2), lambda i: (0, 0)),
            pl.BlockSpec((H2,), lambda i: (0,)),
            pl.BlockSpec((H2,), lambda i: (0,)),
        ],
        out_specs=pl.BlockSpec((B,), lambda i: (i,)),
        out_shape=jax.ShapeDtypeStruct((NK,), jnp.float32),
    )(kemb, k1w, k1b, k2w, k2b, aq)


def _scatter_mean(src, idx, num):
    s = jax.ops.segment_sum(src, idx, num_segments=num)
    c = jax.ops.segment_sum(jnp.ones((src.shape[0],), src.dtype), idx,
                            num_segments=num)
    return s / jnp.clip(c, 1.0)[:, None]


def _conv1d(x, w, b):
    y = jax.lax.conv_general_dilated(x, w, (1,), 'VALID',
                                     dimension_numbers=('NCH', 'OIH', 'NCH'))
    return y + b[None, :, None]


def kernel(n, t, k, m, edge_index0, edge_index1, author_emb, keyword_emb,
           tok_emb, conv2_w, conv2_b, conv3_w, conv3_b, conv4_w, conv4_b,
           cnn_lin_w, cnn_ln_g, cnn_ln_b, q1_w, q1_b, k1_w, k1_b, q2_w, q2_b,
           k2_w, k2_b, g0_l0_w, g0_l0_b, g0_l1_w, g0_l1_b, g0_l2_w, g0_l2_b,
           g0_l3_w, g0_l3_b, g0_ln_g, g0_ln_b, g1_l0_w, g1_l0_b, g1_l1_w,
           g1_l1_b, g1_l2_w, g1_l2_b, g1_l3_w, g1_l3_b, g1_ln_g, g1_ln_b,
           cls_l0_w, cls_l0_b, cls_l1_w, cls_l1_b, cls_l2_w, cls_l2_b,
           cls_l3_w, cls_l3_b, cls_l4_w, cls_l4_b, cls_l5_w, cls_l5_b):
    emb = tok_emb[t]
    xc = jnp.transpose(emb, (0, 2, 1))
    feats = []
    for f, cw, cb in ((2, conv2_w, conv2_b), (3, conv3_w, conv3_b),
                      (4, conv4_w, conv4_b)):
        y = _gelu(_conv1d(xc, cw, cb))
        feats.append(jnp.max(y, axis=-1))
    pc = jnp.concatenate(feats, axis=1)
    pc = pc @ cnn_lin_w
    pc = _gelu(_layernorm(pc, cnn_ln_g, cnn_ln_b))

    aq = (_gelu(pc @ q1_w + q1_b) @ q2_w + q2_b)[0]
    w = _kw_attention(keyword_emb, k1_w, k1_b, k2_w, k2_b, aq)

    xk = keyword_emb[k] * w[k][:, None]
    xk = _scatter_mean(xk, m, N)
    xa = author_emb[n]
    x = jnp.concatenate((xa, xk), axis=-1)

    for (l0w, l0b, l1w, l1b, l2w, l2b, l3w, l3b, lng, lnb, ei) in (
            (g0_l0_w, g0_l0_b, g0_l1_w, g0_l1_b, g0_l2_w, g0_l2_b, g0_l3_w,
             g0_l3_b, g0_ln_g, g0_ln_b, edge_index0),
            (g1_l0_w, g1_l0_b, g1_l1_w, g1_l1_b, g1_l2_w, g1_l2_b, g1_l3_w,
             g1_l3_b, g1_ln_g, g1_ln_b, edge_index1)):
        target = x
        msg = _gelu(x @ l0w + l0b)
        msg = _gelu(msg @ l1w + l1b)
        msg = msg[ei[0]]
        agg = _scatter_mean(msg, ei[1], N)
        comb = jnp.concatenate((target, agg), axis=-1)
        comb = _gelu(comb @ l2w + l2b)
        comb = _gelu(comb @ l3w + l3b)
        comb = comb + target
        x = _layernorm(comb, lng, lnb)

    queries = jnp.tile(_gelu(pc @ cls_l0_w + cls_l0_b), (N - 1, 1))
    edge_from = jnp.tile(_gelu(x[0:1] @ cls_l1_w + cls_l1_b), (N - 1, 1))
    edge_to = _gelu(x[1:] @ cls_l2_w + cls_l2_b)
    z = jnp.concatenate((queries, edge_from, edge_to), axis=-1)
    z = _gelu(z @ cls_l3_w + cls_l3_b)
    z = _gelu(z @ cls_l4_w + cls_l4_b)
    pred = (z @ cls_l5_w + cls_l5_b)[:, 0]
    return (pred, w)
